# baseline (speedup 1.0000x reference)
import functools

import jax
import jax.numpy as jnp
from jax import lax
from jax.experimental import pallas as pl
from jax.experimental.pallas import tpu as pltpu
from jax.experimental.pallas import tpu_sc as plsc

_N = 10000
_E = 320000
_NC = 2
_NS = 16
_NW = _NC * _NS
_CH = 128
_CPT = -(-_E // (_NW * _CH))
_EPAD = _NW * _CPT * _CH
_CPT0 = 127
_CPT1 = 31
_CPTM = max(_CPT0, _CPT1)
assert (_CPT0 + _CPT1) * _NS * _CH >= _E
_RZ = 632
_NA = _NS * _RZ

_R = 1000


def _lrelu(x):
    return jnp.where(x >= 0, x, 0.01 * x)


def _dot(a, b):
    return jnp.dot(a, b, preferred_element_type=jnp.float32)


def _sc_mesh():
    return plsc.VectorSubcoreMesh(core_axis_name="c", subcore_axis_name="s")


def _sc_scatter(h, src3, dst3, D):
    def body(h_hbm, src_hbm, dst_hbm, zeros_hbm, out_hbm,
             src_v, dst_v, rows_v, accum, sem):
        c = lax.axis_index("c")
        s = lax.axis_index("s")
        wid = s * _NC + c
        cpt = jnp.where(c == 0, _CPT0, _CPT1)
        pltpu.sync_copy(zeros_hbm.at[pl.ds(s * _RZ, _RZ)],
                        accum.at[pl.ds(s * _RZ, _RZ)])
        pltpu.sync_copy(src_hbm.at[wid], src_v)
        pltpu.sync_copy(dst_hbm.at[wid], dst_v)
        plsc.subcore_barrier()

        def step(j, carry):
            pltpu.async_copy(h_hbm.at[src_v.at[j]], rows_v, sem).wait()
            pltpu.sync_copy(rows_v, accum.at[dst_v.at[j]], add=True)
            return carry

        lax.fori_loop(0, cpt, step, 0)
        plsc.subcore_barrier()
        pltpu.sync_copy(accum.at[pl.ds(s * _RZ, _RZ)],
                        out_hbm.at[pl.ds(c * _NA + s * _RZ, _RZ)])

    zeros = jnp.zeros((_NA, D), jnp.float32)
    k = pl.kernel(
        body,
        mesh=_sc_mesh(),
        out_type=jax.ShapeDtypeStruct((_NC * _NA, D), jnp.float32),
        scratch_types=[
            pltpu.VMEM((_CPTM, _CH), jnp.int32),
            pltpu.VMEM((_CPTM, _CH), jnp.int32),
            pltpu.VMEM((_CH, D), jnp.float32),
            pltpu.VMEM_SHARED((_NA, D), jnp.float32),
            pltpu.SemaphoreType.DMA,
        ],
    )
    return k(h, src3, dst3, zeros)


def _sc_degree(dst3):
    def body(dst_hbm, ones_hbm, zeros_hbm, out_hbm, dst_v, ones_v, accum):
        c = lax.axis_index("c")
        s = lax.axis_index("s")
        wid = s * _NC + c
        pltpu.sync_copy(zeros_hbm.at[pl.ds(s * _RZ, _RZ)],
                        accum.at[pl.ds(s * _RZ, _RZ)])
        pltpu.sync_copy(dst_hbm.at[wid], dst_v)
        pltpu.sync_copy(ones_hbm, ones_v)
        plsc.subcore_barrier()

        def step(j, carry):
            pltpu.sync_copy(ones_v, accum.at[dst_v.at[j]], add=True)
            return carry

        lax.fori_loop(0, _CPT, step, 0)
        plsc.subcore_barrier()
        pltpu.sync_copy(accum.at[pl.ds(s * _RZ, _RZ)],
                        out_hbm.at[pl.ds(c * _NA + s * _RZ, _RZ)])

    ones = jnp.ones((_CH, 128), jnp.float32)
    zeros = jnp.zeros((_NA, 128), jnp.float32)
    k = pl.kernel(
        body,
        mesh=_sc_mesh(),
        out_type=jax.ShapeDtypeStruct((_NC * _NA, 128), jnp.float32),
        scratch_types=[
            pltpu.VMEM((_CPT, _CH), jnp.int32),
            pltpu.VMEM((_CH, 128), jnp.float32),
            pltpu.VMEM_SHARED((_NA, 128), jnp.float32),
        ],
    )
    return k(dst3, ones, zeros)


def _row_spec(ncols):
    return pl.BlockSpec((_R, ncols), lambda i: (i, 0))


def _full_spec(shape):
    return pl.BlockSpec(shape, lambda i: (0, 0))


def _dis(cnt0_r, cnt1_r):
    deg = cnt0_r[:, :1] + cnt1_r[:, :1] + 1.0
    return lax.rsqrt(deg)


def _tc1_body(des_r, tw_r, np_r, cp_r, c0_r, c1_r,
              Wd, bd, Wt, bt, Wn, bn, Wc, bc, Wi, bi, Wg1, h1_r):
    d = _lrelu(_dot(des_r[...], Wd[...]) + bd[...])
    t = _lrelu(_dot(tw_r[...], Wt[...]) + bt[...])
    n = _lrelu(_dot(np_r[...], Wn[...]) + bn[...])
    cc = _lrelu(_dot(cp_r[...], Wc[...]) + bc[...])
    x = jnp.concatenate([d, t, n, cc], axis=1)
    x0 = _lrelu(_dot(x, Wi[...]) + bi[...])
    dis = _dis(c0_r[...], c1_r[...])
    h1_r[...] = _dot(x0, Wg1[...]) * dis


def _tc2_body(pa_r, pb_r, hp_r, c0_r, c1_r, bg, Wnext, out_r):
    dis = _dis(c0_r[...], c1_r[...])
    x = dis * (pa_r[...] + pb_r[...] + hp_r[...]) + bg[...]
    out_r[...] = _dot(x, Wnext[...]) * dis


def _tc3_body(pa_r, pb_r, hp_r, c0_r, c1_r, bg2, Wo1, bo1, Wg3p, h3_r):
    dis = _dis(c0_r[...], c1_r[...])
    x2 = dis * (pa_r[...] + pb_r[...] + hp_r[...]) + bg2[...]
    y = _lrelu(_dot(x2, Wo1[...]) + bo1[...])
    h3_r[...] = _dot(y, Wg3p[...]) * dis


def _tc4_body(pa_r, pb_r, hp_r, c0_r, c1_r, bg3,
              Wo2, bo2, Wo3, bo3, Wo4, bo4, Wf, bf, out_r):
    dis = _dis(c0_r[...], c1_r[...])
    agg = (pa_r[...] + pb_r[...] + hp_r[...])[:, :64]
    x3 = dis * agg + bg3[...]
    z = _lrelu(_dot(x3, Wo2[...]) + bo2[...])
    z = _lrelu(_dot(z, Wo3[...]) + bo3[...])
    z = _lrelu(_dot(z, Wo4[...]) + bo4[...])
    out_r[...] = _dot(z, Wf[...]) + bf[...]


def _tc_call(body, ins, in_specs, out_cols):
    return pl.pallas_call(
        body,
        grid=(_N // _R,),
        in_specs=in_specs,
        out_specs=_row_spec(out_cols),
        out_shape=jax.ShapeDtypeStruct((_N, out_cols), jnp.float32),
    )(*ins)


def kernel(des, tweet, num_prop, cat_prop, edge_index,
           Wd, bd, Wt, bt, Wn, bn, Wc, bc, Wi, bi,
           Wg1, bg1, Wg2, bg2, Wo1, bo1, Wg3, bg3,
           Wo2, bo2, Wo3, bo3, Wo4, bo4, Wf, bf):
    npad = _EPAD - _E
    dst3s = jnp.concatenate(
        [edge_index[1], jnp.full((npad,), _N, jnp.int32)]).reshape(_NW, _CPT, _CH)

    def asym(v, fill):
        vp = jnp.concatenate([v, jnp.full((npad,), fill, jnp.int32)])
        n0 = _NS * _CPT0 * _CH
        a0 = jnp.pad(vp[:n0].reshape(_NS, _CPT0, _CH),
                     ((0, 0), (0, _CPTM - _CPT0), (0, 0)), constant_values=fill)
        a1 = jnp.pad(vp[n0:].reshape(_NS, _CPT1, _CH),
                     ((0, 0), (0, _CPTM - _CPT1), (0, 0)), constant_values=fill)
        return jnp.stack([a0, a1], axis=1).reshape(_NW, _CPTM, _CH)

    src3 = asym(edge_index[0], 0)
    dst3 = asym(edge_index[1], _N)
    b2 = lambda b: b.reshape(1, -1)

    cnt = _sc_degree(dst3s)
    cnt0 = lax.slice(cnt, (0, 0), (_N, 8))
    cnt1 = lax.slice(cnt, (_NA, 0), (_NA + _N, 8))

    h1 = _tc_call(
        _tc1_body,
        (des, tweet, num_prop, cat_prop, cnt0, cnt1,
         Wd, b2(bd), Wt, b2(bt), Wn, b2(bn), Wc, b2(bc), Wi, b2(bi), Wg1),
        [_row_spec(768), _row_spec(768), _row_spec(5), _row_spec(3),
         _row_spec(8), _row_spec(8),
         _full_spec((768, 32)), _full_spec((1, 32)),
         _full_spec((768, 32)), _full_spec((1, 32)),
         _full_spec((5, 32)), _full_spec((1, 32)),
         _full_spec((3, 32)), _full_spec((1, 32)),
         _full_spec((128, 128)), _full_spec((1, 128)),
         _full_spec((128, 128))],
        128)

    p1 = _sc_scatter(h1, src3, dst3, 128)
    h2 = _tc_call(
        _tc2_body,
        (p1[:_N], p1[_NA:_NA + _N], h1, cnt0, cnt1, b2(bg1), Wg2),
        [_row_spec(128), _row_spec(128), _row_spec(128),
         _row_spec(8), _row_spec(8),
         _full_spec((1, 128)), _full_spec((128, 128))],
        128)

    p2 = _sc_scatter(h2, src3, dst3, 128)
    Wg3p = jnp.pad(Wg3, ((0, 0), (0, 64)))
    h3 = _tc_call(
        _tc3_body,
        (p2[:_N], p2[_NA:_NA + _N], h2, cnt0, cnt1, b2(bg2), Wo1, b2(bo1), Wg3p),
        [_row_spec(128), _row_spec(128), _row_spec(128),
         _row_spec(8), _row_spec(8),
         _full_spec((1, 128)), _full_spec((128, 64)), _full_spec((1, 64)),
         _full_spec((64, 128))],
        128)

    p3 = _sc_scatter(h3, src3, dst3, 128)
    out = _tc_call(
        _tc4_body,
        (p3[:_N], p3[_NA:_NA + _N], h3, cnt0, cnt1, b2(bg3),
         Wo2, b2(bo2), Wo3, b2(bo3), Wo4, b2(bo4), Wf, b2(bf)),
        [_row_spec(128), _row_spec(128), _row_spec(128),
         _row_spec(8), _row_spec(8),
         _full_spec((1, 64)), _full_spec((64, 64)), _full_spec((1, 64)),
         _full_spec((64, 32)), _full_spec((1, 32)),
         _full_spec((32, 16)), _full_spec((1, 16)),
         _full_spec((16, 2)), _full_spec((1, 2))],
        2)
    return out


# speedup vs baseline: 1.0403x; 1.0403x over previous
import functools

import jax
import jax.numpy as jnp
from jax import lax
from jax.experimental import pallas as pl
from jax.experimental.pallas import tpu as pltpu
from jax.experimental.pallas import tpu_sc as plsc

_N = 10000
_E = 320000
_NC = 2
_NS = 16
_NW = _NC * _NS
_CH = 128
_CPT = -(-_E // (_NW * _CH))
_EPAD = _NW * _CPT * _CH
_CPT0 = 119
_CPT1 = 39
_CPTM = max(_CPT0, _CPT1)
assert (_CPT0 + _CPT1) * _NS * _CH >= _E
_RZ = 632
_NA = _NS * _RZ

_R = 1000


def _lrelu(x):
    return jnp.where(x >= 0, x, 0.01 * x)


def _dot(a, b):
    return jnp.dot(a, b, preferred_element_type=jnp.float32)


def _sc_mesh():
    return plsc.VectorSubcoreMesh(core_axis_name="c", subcore_axis_name="s")


def _sc_scatter(h, src3, dst3, D):
    def body(h_hbm, src_hbm, dst_hbm, zeros_hbm, out_hbm,
             src_v, dst_v, rows_v, accum, sem):
        c = lax.axis_index("c")
        s = lax.axis_index("s")
        wid = s * _NC + c
        cpt = jnp.where(c == 0, _CPT0, _CPT1)
        pltpu.sync_copy(zeros_hbm.at[pl.ds(s * _RZ, _RZ)],
                        accum.at[pl.ds(s * _RZ, _RZ)])
        pltpu.sync_copy(src_hbm.at[wid], src_v)
        pltpu.sync_copy(dst_hbm.at[wid], dst_v)
        plsc.subcore_barrier()

        def step(j, carry):
            pltpu.async_copy(h_hbm.at[src_v.at[j]], rows_v, sem).wait()
            pltpu.sync_copy(rows_v, accum.at[dst_v.at[j]], add=True)
            return carry

        lax.fori_loop(0, cpt, step, 0)
        plsc.subcore_barrier()
        pltpu.sync_copy(accum.at[pl.ds(s * _RZ, _RZ)],
                        out_hbm.at[pl.ds(c * _NA + s * _RZ, _RZ)])

    zeros = jnp.zeros((_NA, D), jnp.float32)
    k = pl.kernel(
        body,
        mesh=_sc_mesh(),
        out_type=jax.ShapeDtypeStruct((_NC * _NA, D), jnp.float32),
        scratch_types=[
            pltpu.VMEM((_CPTM, _CH), jnp.int32),
            pltpu.VMEM((_CPTM, _CH), jnp.int32),
            pltpu.VMEM((_CH, D), jnp.float32),
            pltpu.VMEM_SHARED((_NA, D), jnp.float32),
            pltpu.SemaphoreType.DMA,
        ],
    )
    return k(h, src3, dst3, zeros)


def _sc_degree(dst3):
    def body(dst_hbm, ones_hbm, zeros_hbm, out_hbm, dst_v, ones_v, accum):
        c = lax.axis_index("c")
        s = lax.axis_index("s")
        wid = s * _NC + c
        pltpu.sync_copy(zeros_hbm.at[pl.ds(s * _RZ, _RZ)],
                        accum.at[pl.ds(s * _RZ, _RZ)])
        pltpu.sync_copy(dst_hbm.at[wid], dst_v)
        pltpu.sync_copy(ones_hbm, ones_v)
        plsc.subcore_barrier()

        def step(j, carry):
            pltpu.sync_copy(ones_v, accum.at[dst_v.at[j]], add=True)
            return carry

        lax.fori_loop(0, _CPT, step, 0)
        plsc.subcore_barrier()
        pltpu.sync_copy(accum.at[pl.ds(s * _RZ, _RZ)],
                        out_hbm.at[pl.ds(c * _NA + s * _RZ, _RZ)])

    ones = jnp.ones((_CH, 128), jnp.float32)
    zeros = jnp.zeros((_NA, 128), jnp.float32)
    k = pl.kernel(
        body,
        mesh=_sc_mesh(),
        out_type=jax.ShapeDtypeStruct((_NC * _NA, 128), jnp.float32),
        scratch_types=[
            pltpu.VMEM((_CPT, _CH), jnp.int32),
            pltpu.VMEM((_CH, 128), jnp.float32),
            pltpu.VMEM_SHARED((_NA, 128), jnp.float32),
        ],
    )
    return k(dst3, ones, zeros)


def _row_spec(ncols):
    return pl.BlockSpec((_R, ncols), lambda i: (i, 0))


def _full_spec(shape):
    return pl.BlockSpec(shape, lambda i: (0, 0))


def _dis(cnt0_r, cnt1_r):
    deg = cnt0_r[:, :1] + cnt1_r[:, :1] + 1.0
    return lax.rsqrt(deg)


def _tc1_body(des_r, tw_r, np_r, cp_r, c0_r, c1_r,
              Wd, bd, Wt, bt, Wn, bn, Wc, bc, Wi, bi, Wg1, h1_r):
    d = _lrelu(_dot(des_r[...], Wd[...]) + bd[...])
    t = _lrelu(_dot(tw_r[...], Wt[...]) + bt[...])
    n = _lrelu(_dot(np_r[...], Wn[...]) + bn[...])
    cc = _lrelu(_dot(cp_r[...], Wc[...]) + bc[...])
    x = jnp.concatenate([d, t, n, cc], axis=1)
    x0 = _lrelu(_dot(x, Wi[...]) + bi[...])
    dis = _dis(c0_r[...], c1_r[...])
    h1_r[...] = _dot(x0, Wg1[...]) * dis


def _tc2_body(pa_r, pb_r, hp_r, c0_r, c1_r, bg, Wnext, out_r):
    dis = _dis(c0_r[...], c1_r[...])
    x = dis * (pa_r[...] + pb_r[...] + hp_r[...]) + bg[...]
    out_r[...] = _dot(x, Wnext[...]) * dis


def _tc3_body(pa_r, pb_r, hp_r, c0_r, c1_r, bg2, Wo1, bo1, Wg3p, h3_r):
    dis = _dis(c0_r[...], c1_r[...])
    x2 = dis * (pa_r[...] + pb_r[...] + hp_r[...]) + bg2[...]
    y = _lrelu(_dot(x2, Wo1[...]) + bo1[...])
    h3_r[...] = _dot(y, Wg3p[...]) * dis


def _tc4_body(pa_r, pb_r, hp_r, c0_r, c1_r, bg3,
              Wo2, bo2, Wo3, bo3, Wo4, bo4, Wf, bf, out_r):
    dis = _dis(c0_r[...], c1_r[...])
    agg = (pa_r[...] + pb_r[...] + hp_r[...])[:, :64]
    x3 = dis * agg + bg3[...]
    z = _lrelu(_dot(x3, Wo2[...]) + bo2[...])
    z = _lrelu(_dot(z, Wo3[...]) + bo3[...])
    z = _lrelu(_dot(z, Wo4[...]) + bo4[...])
    out_r[...] = _dot(z, Wf[...]) + bf[...]


def _tc_call(body, ins, in_specs, out_cols):
    return pl.pallas_call(
        body,
        grid=(_N // _R,),
        in_specs=in_specs,
        out_specs=_row_spec(out_cols),
        out_shape=jax.ShapeDtypeStruct((_N, out_cols), jnp.float32),
    )(*ins)


def kernel(des, tweet, num_prop, cat_prop, edge_index,
           Wd, bd, Wt, bt, Wn, bn, Wc, bc, Wi, bi,
           Wg1, bg1, Wg2, bg2, Wo1, bo1, Wg3, bg3,
           Wo2, bo2, Wo3, bo3, Wo4, bo4, Wf, bf):
    npad = _EPAD - _E
    dst3s = jnp.concatenate(
        [edge_index[1], jnp.full((npad,), _N, jnp.int32)]).reshape(_NW, _CPT, _CH)

    def asym(v, fill):
        vp = jnp.concatenate([v, jnp.full((npad,), fill, jnp.int32)])
        n0 = _NS * _CPT0 * _CH
        a0 = jnp.pad(vp[:n0].reshape(_NS, _CPT0, _CH),
                     ((0, 0), (0, _CPTM - _CPT0), (0, 0)), constant_values=fill)
        a1 = jnp.pad(vp[n0:].reshape(_NS, _CPT1, _CH),
                     ((0, 0), (0, _CPTM - _CPT1), (0, 0)), constant_values=fill)
        return jnp.stack([a0, a1], axis=1).reshape(_NW, _CPTM, _CH)

    src3 = asym(edge_index[0], 0)
    dst3 = asym(edge_index[1], _N)
    b2 = lambda b: b.reshape(1, -1)

    cnt = _sc_degree(dst3s)
    cnt0 = lax.slice(cnt, (0, 0), (_N, 8))
    cnt1 = lax.slice(cnt, (_NA, 0), (_NA + _N, 8))

    h1 = _tc_call(
        _tc1_body,
        (des, tweet, num_prop, cat_prop, cnt0, cnt1,
         Wd, b2(bd), Wt, b2(bt), Wn, b2(bn), Wc, b2(bc), Wi, b2(bi), Wg1),
        [_row_spec(768), _row_spec(768), _row_spec(5), _row_spec(3),
         _row_spec(8), _row_spec(8),
         _full_spec((768, 32)), _full_spec((1, 32)),
         _full_spec((768, 32)), _full_spec((1, 32)),
         _full_spec((5, 32)), _full_spec((1, 32)),
         _full_spec((3, 32)), _full_spec((1, 32)),
         _full_spec((128, 128)), _full_spec((1, 128)),
         _full_spec((128, 128))],
        128)

    p1 = _sc_scatter(h1, src3, dst3, 128)
    h2 = _tc_call(
        _tc2_body,
        (p1[:_N], p1[_NA:_NA + _N], h1, cnt0, cnt1, b2(bg1), Wg2),
        [_row_spec(128), _row_spec(128), _row_spec(128),
         _row_spec(8), _row_spec(8),
         _full_spec((1, 128)), _full_spec((128, 128))],
        128)

    p2 = _sc_scatter(h2, src3, dst3, 128)
    Wg3p = jnp.pad(Wg3, ((0, 0), (0, 64)))
    h3 = _tc_call(
        _tc3_body,
        (p2[:_N], p2[_NA:_NA + _N], h2, cnt0, cnt1, b2(bg2), Wo1, b2(bo1), Wg3p),
        [_row_spec(128), _row_spec(128), _row_spec(128),
         _row_spec(8), _row_spec(8),
         _full_spec((1, 128)), _full_spec((128, 64)), _full_spec((1, 64)),
         _full_spec((64, 128))],
        128)

    p3 = _sc_scatter(h3, src3, dst3, 128)
    out = _tc_call(
        _tc4_body,
        (p3[:_N], p3[_NA:_NA + _N], h3, cnt0, cnt1, b2(bg3),
         Wo2, b2(bo2), Wo3, b2(bo3), Wo4, b2(bo4), Wf, b2(bf)),
        [_row_spec(128), _row_spec(128), _row_spec(128),
         _row_spec(8), _row_spec(8),
         _full_spec((1, 64)), _full_spec((64, 64)), _full_spec((1, 64)),
         _full_spec((64, 32)), _full_spec((1, 32)),
         _full_spec((32, 16)), _full_spec((1, 16)),
         _full_spec((16, 2)), _full_spec((1, 2))],
        2)
    return out


# speedup vs baseline: 1.1014x; 1.0588x over previous
import functools

import jax
import jax.numpy as jnp
from jax import lax
from jax.experimental import pallas as pl
from jax.experimental.pallas import tpu as pltpu
from jax.experimental.pallas import tpu_sc as plsc

_N = 10000
_E = 320000
_NC = 2
_NS = 16
_NW = _NC * _NS
_CH = 128
_CPT = -(-_E // (_NW * _CH))
_EPAD = _NW * _CPT * _CH
_CPT0 = 111
_CPT1 = 47
_CPH = (_CPT0 + 1) // 2
_CPTM = max(_CPT0, _CPT1)
assert (_CPT0 + _CPT1) * _NS * _CH >= _E
_RZ = 632
_NA = _NS * _RZ

_R = 1000


def _lrelu(x):
    return jnp.where(x >= 0, x, 0.01 * x)


def _dot(a, b):
    return jnp.dot(a, b, preferred_element_type=jnp.float32)


def _sc_mesh():
    return plsc.VectorSubcoreMesh(core_axis_name="c", subcore_axis_name="s")


def _sc_scatter(h, se3, so3, de3, do3, D):
    def body(h_hbm, se_hbm, so_hbm, de_hbm, do_hbm, zeros_hbm, out_hbm,
             src_v, dst_v, rows_a, rows_b, accum, sem_a, sem_b):
        c = lax.axis_index("c")
        s = lax.axis_index("s")
        wid = s * _NC + c
        cpt = jnp.where(c == 0, _CPT0, _CPT1)
        pltpu.sync_copy(zeros_hbm.at[pl.ds(s * _RZ, _RZ)],
                        accum.at[pl.ds(s * _RZ, _RZ)])
        plsc.subcore_barrier()

        def drain(buf, sem):
            pltpu.make_async_copy(h_hbm.at[pl.ds(0, _CH)], buf, sem).wait()

        def phase(src_hbm, dst_hbm, n):
            pltpu.sync_copy(src_hbm.at[wid], src_v)
            pltpu.sync_copy(dst_hbm.at[wid], dst_v)
            pltpu.async_copy(h_hbm.at[src_v.at[0]], rows_a, sem_a)

            def step2(i, carry):
                ja = 2 * i
                pltpu.async_copy(h_hbm.at[src_v.at[ja + 1]], rows_b, sem_b)
                drain(rows_a, sem_a)
                pltpu.sync_copy(rows_a, accum.at[dst_v.at[ja]], add=True)
                jn = jnp.minimum(ja + 2, n - 1)
                pltpu.async_copy(h_hbm.at[src_v.at[jn]], rows_a, sem_a)
                drain(rows_b, sem_b)
                pltpu.sync_copy(rows_b, accum.at[dst_v.at[ja + 1]], add=True)
                return carry

            lax.fori_loop(0, n // 2, step2, 0)
            drain(rows_a, sem_a)

            @pl.when(n % 2 == 1)
            def _():
                pltpu.sync_copy(rows_a, accum.at[dst_v.at[n - 1]], add=True)

        phase(se_hbm, de_hbm, (cpt + 1) // 2)
        phase(so_hbm, do_hbm, cpt // 2)
        plsc.subcore_barrier()
        pltpu.sync_copy(accum.at[pl.ds(s * _RZ, _RZ)],
                        out_hbm.at[pl.ds(c * _NA + s * _RZ, _RZ)])

    zeros = jnp.zeros((_NA, D), jnp.float32)
    k = pl.kernel(
        body,
        mesh=_sc_mesh(),
        out_type=jax.ShapeDtypeStruct((_NC * _NA, D), jnp.float32),
        scratch_types=[
            pltpu.VMEM((_CPH, _CH), jnp.int32),
            pltpu.VMEM((_CPH, _CH), jnp.int32),
            pltpu.VMEM((_CH, D), jnp.float32),
            pltpu.VMEM((_CH, D), jnp.float32),
            pltpu.VMEM_SHARED((_NA, D), jnp.float32),
            pltpu.SemaphoreType.DMA,
            pltpu.SemaphoreType.DMA,
        ],
    )
    return k(h, se3, so3, de3, do3, zeros)


def _sc_degree(dst3):
    def body(dst_hbm, ones_hbm, zeros_hbm, out_hbm, dst_v, ones_v, accum):
        c = lax.axis_index("c")
        s = lax.axis_index("s")
        wid = s * _NC + c
        pltpu.sync_copy(zeros_hbm.at[pl.ds(s * _RZ, _RZ)],
                        accum.at[pl.ds(s * _RZ, _RZ)])
        pltpu.sync_copy(dst_hbm.at[wid], dst_v)
        pltpu.sync_copy(ones_hbm, ones_v)
        plsc.subcore_barrier()

        def step(j, carry):
            pltpu.sync_copy(ones_v, accum.at[dst_v.at[j]], add=True)
            return carry

        lax.fori_loop(0, _CPT, step, 0)
        plsc.subcore_barrier()
        pltpu.sync_copy(accum.at[pl.ds(s * _RZ, _RZ)],
                        out_hbm.at[pl.ds(c * _NA + s * _RZ, _RZ)])

    ones = jnp.ones((_CH, 128), jnp.float32)
    zeros = jnp.zeros((_NA, 128), jnp.float32)
    k = pl.kernel(
        body,
        mesh=_sc_mesh(),
        out_type=jax.ShapeDtypeStruct((_NC * _NA, 128), jnp.float32),
        scratch_types=[
            pltpu.VMEM((_CPT, _CH), jnp.int32),
            pltpu.VMEM((_CH, 128), jnp.float32),
            pltpu.VMEM_SHARED((_NA, 128), jnp.float32),
        ],
    )
    return k(dst3, ones, zeros)


def _row_spec(ncols):
    return pl.BlockSpec((_R, ncols), lambda i: (i, 0))


def _full_spec(shape):
    return pl.BlockSpec(shape, lambda i: (0, 0))


def _dis(cnt0_r, cnt1_r):
    deg = cnt0_r[:, :1] + cnt1_r[:, :1] + 1.0
    return lax.rsqrt(deg)


def _tc1_body(des_r, tw_r, np_r, cp_r, c0_r, c1_r,
              Wd, bd, Wt, bt, Wn, bn, Wc, bc, Wi, bi, Wg1, h1_r):
    d = _lrelu(_dot(des_r[...], Wd[...]) + bd[...])
    t = _lrelu(_dot(tw_r[...], Wt[...]) + bt[...])
    n = _lrelu(_dot(np_r[...], Wn[...]) + bn[...])
    cc = _lrelu(_dot(cp_r[...], Wc[...]) + bc[...])
    x = jnp.concatenate([d, t, n, cc], axis=1)
    x0 = _lrelu(_dot(x, Wi[...]) + bi[...])
    dis = _dis(c0_r[...], c1_r[...])
    h1_r[...] = _dot(x0, Wg1[...]) * dis


def _tc2_body(pa_r, pb_r, hp_r, c0_r, c1_r, bg, Wnext, out_r):
    dis = _dis(c0_r[...], c1_r[...])
    x = dis * (pa_r[...] + pb_r[...] + hp_r[...]) + bg[...]
    out_r[...] = _dot(x, Wnext[...]) * dis


def _tc3_body(pa_r, pb_r, hp_r, c0_r, c1_r, bg2, Wo1, bo1, Wg3p, h3_r):
    dis = _dis(c0_r[...], c1_r[...])
    x2 = dis * (pa_r[...] + pb_r[...] + hp_r[...]) + bg2[...]
    y = _lrelu(_dot(x2, Wo1[...]) + bo1[...])
    h3_r[...] = _dot(y, Wg3p[...]) * dis


def _tc4_body(pa_r, pb_r, hp_r, c0_r, c1_r, bg3,
              Wo2, bo2, Wo3, bo3, Wo4, bo4, Wf, bf, out_r):
    dis = _dis(c0_r[...], c1_r[...])
    agg = (pa_r[...] + pb_r[...] + hp_r[...])[:, :64]
    x3 = dis * agg + bg3[...]
    z = _lrelu(_dot(x3, Wo2[...]) + bo2[...])
    z = _lrelu(_dot(z, Wo3[...]) + bo3[...])
    z = _lrelu(_dot(z, Wo4[...]) + bo4[...])
    out_r[...] = _dot(z, Wf[...]) + bf[...]


def _tc_call(body, ins, in_specs, out_cols):
    return pl.pallas_call(
        body,
        grid=(_N // _R,),
        in_specs=in_specs,
        out_specs=_row_spec(out_cols),
        out_shape=jax.ShapeDtypeStruct((_N, out_cols), jnp.float32),
    )(*ins)


def kernel(des, tweet, num_prop, cat_prop, edge_index,
           Wd, bd, Wt, bt, Wn, bn, Wc, bc, Wi, bi,
           Wg1, bg1, Wg2, bg2, Wo1, bo1, Wg3, bg3,
           Wo2, bo2, Wo3, bo3, Wo4, bo4, Wf, bf):
    npad = _EPAD - _E
    dst3s = jnp.concatenate(
        [edge_index[1], jnp.full((npad,), _N, jnp.int32)]).reshape(_NW, _CPT, _CH)

    def asym(v, fill):
        vp = jnp.concatenate([v, jnp.full((npad,), fill, jnp.int32)])
        n0 = _NS * _CPT0 * _CH
        a0 = jnp.pad(vp[:n0].reshape(_NS, _CPT0, _CH),
                     ((0, 0), (0, _CPTM - _CPT0), (0, 0)), constant_values=fill)
        a1 = jnp.pad(vp[n0:].reshape(_NS, _CPT1, _CH),
                     ((0, 0), (0, _CPTM - _CPT1), (0, 0)), constant_values=fill)
        return jnp.stack([a0, a1], axis=1).reshape(_NW, _CPTM, _CH)

    src3 = asym(edge_index[0], 0)
    dst3 = asym(edge_index[1], _N)
    pad1 = ((0, 0), (0, 1), (0, 0))
    se3 = src3[:, 0::2]
    so3 = jnp.pad(src3[:, 1::2], pad1)
    de3 = dst3[:, 0::2]
    do3 = jnp.pad(dst3[:, 1::2], pad1, constant_values=_N)
    b2 = lambda b: b.reshape(1, -1)

    cnt = _sc_degree(dst3s)
    cnt0 = lax.slice(cnt, (0, 0), (_N, 8))
    cnt1 = lax.slice(cnt, (_NA, 0), (_NA + _N, 8))

    h1 = _tc_call(
        _tc1_body,
        (des, tweet, num_prop, cat_prop, cnt0, cnt1,
         Wd, b2(bd), Wt, b2(bt), Wn, b2(bn), Wc, b2(bc), Wi, b2(bi), Wg1),
        [_row_spec(768), _row_spec(768), _row_spec(5), _row_spec(3),
         _row_spec(8), _row_spec(8),
         _full_spec((768, 32)), _full_spec((1, 32)),
         _full_spec((768, 32)), _full_spec((1, 32)),
         _full_spec((5, 32)), _full_spec((1, 32)),
         _full_spec((3, 32)), _full_spec((1, 32)),
         _full_spec((128, 128)), _full_spec((1, 128)),
         _full_spec((128, 128))],
        128)

    p1 = _sc_scatter(h1, se3, so3, de3, do3, 128)
    h2 = _tc_call(
        _tc2_body,
        (p1[:_N], p1[_NA:_NA + _N], h1, cnt0, cnt1, b2(bg1), Wg2),
        [_row_spec(128), _row_spec(128), _row_spec(128),
         _row_spec(8), _row_spec(8),
         _full_spec((1, 128)), _full_spec((128, 128))],
        128)

    p2 = _sc_scatter(h2, se3, so3, de3, do3, 128)
    Wg3p = jnp.pad(Wg3, ((0, 0), (0, 64)))
    h3 = _tc_call(
        _tc3_body,
        (p2[:_N], p2[_NA:_NA + _N], h2, cnt0, cnt1, b2(bg2), Wo1, b2(bo1), Wg3p),
        [_row_spec(128), _row_spec(128), _row_spec(128),
         _row_spec(8), _row_spec(8),
         _full_spec((1, 128)), _full_spec((128, 64)), _full_spec((1, 64)),
         _full_spec((64, 128))],
        128)

    p3 = _sc_scatter(h3, se3, so3, de3, do3, 128)
    out = _tc_call(
        _tc4_body,
        (p3[:_N], p3[_NA:_NA + _N], h3, cnt0, cnt1, b2(bg3),
         Wo2, b2(bo2), Wo3, b2(bo3), Wo4, b2(bo4), Wf, b2(bf)),
        [_row_spec(128), _row_spec(128), _row_spec(128),
         _row_spec(8), _row_spec(8),
         _full_spec((1, 64)), _full_spec((64, 64)), _full_spec((1, 64)),
         _full_spec((64, 32)), _full_spec((1, 32)),
         _full_spec((32, 16)), _full_spec((1, 16)),
         _full_spec((16, 2)), _full_spec((1, 2))],
        2)
    return out


# speedup vs baseline: 1.1682x; 1.0607x over previous
import functools

import jax
import jax.numpy as jnp
from jax import lax
from jax.experimental import pallas as pl
from jax.experimental.pallas import tpu as pltpu
from jax.experimental.pallas import tpu_sc as plsc

_N = 10000
_E = 320000
_NC = 2
_NS = 16
_NW = _NC * _NS
_CH = 128
_CPT = -(-_E // (_NW * _CH))
_EPAD = _NW * _CPT * _CH
_CPT0 = 125
_CPT1 = 33
_CPH = (_CPT0 + 1) // 2
_CPTM = max(_CPT0, _CPT1)
assert (_CPT0 + _CPT1) * _NS * _CH >= _E
_RZ = 632
_NA = _NS * _RZ

_R = 1000


def _lrelu(x):
    return jnp.where(x >= 0, x, 0.01 * x)


def _dot(a, b):
    return jnp.dot(a, b, preferred_element_type=jnp.float32)


def _sc_mesh():
    return plsc.VectorSubcoreMesh(core_axis_name="c", subcore_axis_name="s")


def _sc_scatter(h, se3, so3, de3, do3, D):
    def body(h_hbm, se_hbm, so_hbm, de_hbm, do_hbm, zeros_hbm, out_hbm,
             src_v, dst_v, rows_a, rows_b, accum, sem_a, sem_b):
        c = lax.axis_index("c")
        s = lax.axis_index("s")
        wid = s * _NC + c
        cpt = jnp.where(c == 0, _CPT0, _CPT1)
        pltpu.sync_copy(zeros_hbm.at[pl.ds(s * _RZ, _RZ)],
                        accum.at[pl.ds(s * _RZ, _RZ)])
        plsc.subcore_barrier()

        def drain(buf, sem):
            pltpu.make_async_copy(h_hbm.at[pl.ds(0, _CH)], buf, sem).wait()

        def phase(src_hbm, dst_hbm, n):
            pltpu.sync_copy(src_hbm.at[wid], src_v)
            pltpu.sync_copy(dst_hbm.at[wid], dst_v)
            pltpu.async_copy(h_hbm.at[src_v.at[0]], rows_a, sem_a)

            def step2(i, carry):
                ja = 2 * i
                pltpu.async_copy(h_hbm.at[src_v.at[ja + 1]], rows_b, sem_b)
                drain(rows_a, sem_a)
                pltpu.sync_copy(rows_a, accum.at[dst_v.at[ja]], add=True)
                jn = jnp.minimum(ja + 2, n - 1)
                pltpu.async_copy(h_hbm.at[src_v.at[jn]], rows_a, sem_a)
                drain(rows_b, sem_b)
                pltpu.sync_copy(rows_b, accum.at[dst_v.at[ja + 1]], add=True)
                return carry

            lax.fori_loop(0, n // 2, step2, 0)
            drain(rows_a, sem_a)

            @pl.when(n % 2 == 1)
            def _():
                pltpu.sync_copy(rows_a, accum.at[dst_v.at[n - 1]], add=True)

        phase(se_hbm, de_hbm, (cpt + 1) // 2)
        phase(so_hbm, do_hbm, cpt // 2)
        plsc.subcore_barrier()
        pltpu.sync_copy(accum.at[pl.ds(s * _RZ, _RZ)],
                        out_hbm.at[pl.ds(c * _NA + s * _RZ, _RZ)])

    zeros = jnp.zeros((_NA, D), jnp.float32)
    k = pl.kernel(
        body,
        mesh=_sc_mesh(),
        out_type=jax.ShapeDtypeStruct((_NC * _NA, D), jnp.float32),
        scratch_types=[
            pltpu.VMEM((_CPH, _CH), jnp.int32),
            pltpu.VMEM((_CPH, _CH), jnp.int32),
            pltpu.VMEM((_CH, D), jnp.float32),
            pltpu.VMEM((_CH, D), jnp.float32),
            pltpu.VMEM_SHARED((_NA, D), jnp.float32),
            pltpu.SemaphoreType.DMA,
            pltpu.SemaphoreType.DMA,
        ],
    )
    return k(h, se3, so3, de3, do3, zeros)


def _sc_degree(dst3):
    def body(dst_hbm, ones_hbm, zeros_hbm, out_hbm, dst_v, ones_v, accum):
        c = lax.axis_index("c")
        s = lax.axis_index("s")
        wid = s * _NC + c
        pltpu.sync_copy(zeros_hbm.at[pl.ds(s * _RZ, _RZ)],
                        accum.at[pl.ds(s * _RZ, _RZ)])
        pltpu.sync_copy(dst_hbm.at[wid], dst_v)
        pltpu.sync_copy(ones_hbm, ones_v)
        plsc.subcore_barrier()

        def step(j, carry):
            pltpu.sync_copy(ones_v, accum.at[dst_v.at[j]], add=True)
            return carry

        lax.fori_loop(0, _CPT, step, 0)
        plsc.subcore_barrier()
        pltpu.sync_copy(accum.at[pl.ds(s * _RZ, _RZ)],
                        out_hbm.at[pl.ds(c * _NA + s * _RZ, _RZ)])

    ones = jnp.ones((_CH, 128), jnp.float32)
    zeros = jnp.zeros((_NA, 128), jnp.float32)
    k = pl.kernel(
        body,
        mesh=_sc_mesh(),
        out_type=jax.ShapeDtypeStruct((_NC * _NA, 128), jnp.float32),
        scratch_types=[
            pltpu.VMEM((_CPT, _CH), jnp.int32),
            pltpu.VMEM((_CH, 128), jnp.float32),
            pltpu.VMEM_SHARED((_NA, 128), jnp.float32),
        ],
    )
    return k(dst3, ones, zeros)


def _row_spec(ncols):
    return pl.BlockSpec((_R, ncols), lambda i: (i, 0))


def _full_spec(shape):
    return pl.BlockSpec(shape, lambda i: (0, 0))


def _dis(cnt0_r, cnt1_r):
    deg = cnt0_r[:, :1] + cnt1_r[:, :1] + 1.0
    return lax.rsqrt(deg)


def _tc1_body(des_r, tw_r, np_r, cp_r, c0_r, c1_r,
              Wd, bd, Wt, bt, Wn, bn, Wc, bc, Wi, bi, Wg1, h1_r):
    d = _lrelu(_dot(des_r[...], Wd[...]) + bd[...])
    t = _lrelu(_dot(tw_r[...], Wt[...]) + bt[...])
    n = _lrelu(_dot(np_r[...], Wn[...]) + bn[...])
    cc = _lrelu(_dot(cp_r[...], Wc[...]) + bc[...])
    x = jnp.concatenate([d, t, n, cc], axis=1)
    x0 = _lrelu(_dot(x, Wi[...]) + bi[...])
    dis = _dis(c0_r[...], c1_r[...])
    h1_r[...] = _dot(x0, Wg1[...]) * dis


def _tc2_body(pa_r, pb_r, hp_r, c0_r, c1_r, bg, Wnext, out_r):
    dis = _dis(c0_r[...], c1_r[...])
    x = dis * (pa_r[...] + pb_r[...] + hp_r[...]) + bg[...]
    out_r[...] = _dot(x, Wnext[...]) * dis


def _tc3_body(pa_r, pb_r, hp_r, c0_r, c1_r, bg2, Wo1, bo1, Wg3p, h3_r):
    dis = _dis(c0_r[...], c1_r[...])
    x2 = dis * (pa_r[...] + pb_r[...] + hp_r[...]) + bg2[...]
    y = _lrelu(_dot(x2, Wo1[...]) + bo1[...])
    h3_r[...] = _dot(y, Wg3p[...]) * dis


def _tc4_body(pa_r, pb_r, hp_r, c0_r, c1_r, bg3,
              Wo2, bo2, Wo3, bo3, Wo4, bo4, Wf, bf, out_r):
    dis = _dis(c0_r[...], c1_r[...])
    agg = (pa_r[...] + pb_r[...] + hp_r[...])[:, :64]
    x3 = dis * agg + bg3[...]
    z = _lrelu(_dot(x3, Wo2[...]) + bo2[...])
    z = _lrelu(_dot(z, Wo3[...]) + bo3[...])
    z = _lrelu(_dot(z, Wo4[...]) + bo4[...])
    out_r[...] = _dot(z, Wf[...]) + bf[...]


def _tc_call(body, ins, in_specs, out_cols):
    return pl.pallas_call(
        body,
        grid=(_N // _R,),
        in_specs=in_specs,
        out_specs=_row_spec(out_cols),
        out_shape=jax.ShapeDtypeStruct((_N, out_cols), jnp.float32),
    )(*ins)


def kernel(des, tweet, num_prop, cat_prop, edge_index,
           Wd, bd, Wt, bt, Wn, bn, Wc, bc, Wi, bi,
           Wg1, bg1, Wg2, bg2, Wo1, bo1, Wg3, bg3,
           Wo2, bo2, Wo3, bo3, Wo4, bo4, Wf, bf):
    npad = _EPAD - _E
    dst3s = jnp.concatenate(
        [edge_index[1], jnp.full((npad,), _N, jnp.int32)]).reshape(_NW, _CPT, _CH)

    def asym(v, fill):
        vp = jnp.concatenate([v, jnp.full((npad,), fill, jnp.int32)])
        n0 = _NS * _CPT0 * _CH
        a0 = jnp.pad(vp[:n0].reshape(_NS, _CPT0, _CH),
                     ((0, 0), (0, _CPTM - _CPT0), (0, 0)), constant_values=fill)
        a1 = jnp.pad(vp[n0:].reshape(_NS, _CPT1, _CH),
                     ((0, 0), (0, _CPTM - _CPT1), (0, 0)), constant_values=fill)
        return jnp.stack([a0, a1], axis=1).reshape(_NW, _CPTM, _CH)

    src3 = asym(edge_index[0], 0)
    dst3 = asym(edge_index[1], _N)
    pad1 = ((0, 0), (0, 1), (0, 0))
    se3 = src3[:, 0::2]
    so3 = jnp.pad(src3[:, 1::2], pad1)
    de3 = dst3[:, 0::2]
    do3 = jnp.pad(dst3[:, 1::2], pad1, constant_values=_N)
    b2 = lambda b: b.reshape(1, -1)

    cnt = _sc_degree(dst3s)
    cnt0 = lax.slice(cnt, (0, 0), (_N, 8))
    cnt1 = lax.slice(cnt, (_NA, 0), (_NA + _N, 8))

    h1 = _tc_call(
        _tc1_body,
        (des, tweet, num_prop, cat_prop, cnt0, cnt1,
         Wd, b2(bd), Wt, b2(bt), Wn, b2(bn), Wc, b2(bc), Wi, b2(bi), Wg1),
        [_row_spec(768), _row_spec(768), _row_spec(5), _row_spec(3),
         _row_spec(8), _row_spec(8),
         _full_spec((768, 32)), _full_spec((1, 32)),
         _full_spec((768, 32)), _full_spec((1, 32)),
         _full_spec((5, 32)), _full_spec((1, 32)),
         _full_spec((3, 32)), _full_spec((1, 32)),
         _full_spec((128, 128)), _full_spec((1, 128)),
         _full_spec((128, 128))],
        128)

    p1 = _sc_scatter(h1, se3, so3, de3, do3, 128)
    h2 = _tc_call(
        _tc2_body,
        (p1[:_N], p1[_NA:_NA + _N], h1, cnt0, cnt1, b2(bg1), Wg2),
        [_row_spec(128), _row_spec(128), _row_spec(128),
         _row_spec(8), _row_spec(8),
         _full_spec((1, 128)), _full_spec((128, 128))],
        128)

    p2 = _sc_scatter(h2, se3, so3, de3, do3, 128)
    Wg3p = jnp.pad(Wg3, ((0, 0), (0, 64)))
    h3 = _tc_call(
        _tc3_body,
        (p2[:_N], p2[_NA:_NA + _N], h2, cnt0, cnt1, b2(bg2), Wo1, b2(bo1), Wg3p),
        [_row_spec(128), _row_spec(128), _row_spec(128),
         _row_spec(8), _row_spec(8),
         _full_spec((1, 128)), _full_spec((128, 64)), _full_spec((1, 64)),
         _full_spec((64, 128))],
        128)

    p3 = _sc_scatter(h3, se3, so3, de3, do3, 128)
    out = _tc_call(
        _tc4_body,
        (p3[:_N], p3[_NA:_NA + _N], h3, cnt0, cnt1, b2(bg3),
         Wo2, b2(bo2), Wo3, b2(bo3), Wo4, b2(bo4), Wf, b2(bf)),
        [_row_spec(128), _row_spec(128), _row_spec(128),
         _row_spec(8), _row_spec(8),
         _full_spec((1, 64)), _full_spec((64, 64)), _full_spec((1, 64)),
         _full_spec((64, 32)), _full_spec((1, 32)),
         _full_spec((32, 16)), _full_spec((1, 16)),
         _full_spec((16, 2)), _full_spec((1, 2))],
        2)
    return out
